# baseline (speedup 1.0000x reference)
import functools

import jax
import jax.numpy as jnp
from jax.experimental import pallas as pl

EMBED_DIM = 768
HIDDEN_DIM = 768
NUM_EXPERTS = 8
TOP_K = 2


def _router_kernel(x_ref, wg_ref, wcomb_ref, aux_ref):
    x = x_ref[...]
    wg = wg_ref[...]
    logits = jax.lax.dot_general(
        x, wg, (((1,), (1,)), ((), ())), preferred_element_type=jnp.float32
    )
    m = jnp.max(logits, axis=-1, keepdims=True)
    ex = jnp.exp(logits - m)
    probs = ex / jnp.sum(ex, axis=-1, keepdims=True)

    T, E = probs.shape
    idx = jax.lax.broadcasted_iota(jnp.int32, (T, E), 1)
    big = jnp.int32(E)
    m1 = jnp.max(probs, axis=-1, keepdims=True)
    i1 = jnp.min(jnp.where(probs == m1, idx, big), axis=-1, keepdims=True)
    masked = jnp.where(idx == i1, -jnp.inf, probs)
    m2 = jnp.max(masked, axis=-1, keepdims=True)
    i2 = jnp.min(jnp.where(masked == m2, idx, big), axis=-1, keepdims=True)

    denom = m1 + m2
    g1 = m1 / denom
    g2 = m2 / denom

    onehot1 = (idx == i1).astype(jnp.float32)
    onehot2 = (idx == i2).astype(jnp.float32)
    wcomb = g1 * onehot1 + g2 * onehot2
    wcomb_ref[...] = jnp.transpose(wcomb)

    f = jnp.sum(onehot1 + onehot2, axis=0) / jnp.float32(T)
    p = jnp.sum(probs, axis=0) / jnp.float32(T)
    aux_ref[...] = (jnp.float32(NUM_EXPERTS) * jnp.sum(f * p)).reshape(1, 1)


def _ffn_kernel(x_ref, w1_ref, b1_ref, w2_ref, b2_ref, wt_ref, out_ref):
    e = pl.program_id(1)

    @pl.when(e == 0)
    def _init():
        out_ref[...] = jnp.zeros_like(out_ref)

    xb = x_ref[...]
    w1 = w1_ref[0]
    h = jax.lax.dot_general(
        xb, w1, (((1,), (1,)), ((), ())), preferred_element_type=jnp.float32
    ) + b1_ref[0]
    h = h * jax.nn.sigmoid(h)
    w2 = w2_ref[0]
    eo = jax.lax.dot_general(
        h.astype(w2.dtype), w2, (((1,), (1,)), ((), ())),
        preferred_element_type=jnp.float32,
    ) + b2_ref[0]
    wcol = jnp.transpose(wt_ref[pl.ds(e, 1), :])
    out_ref[...] += wcol * eo


def kernel(x, Wg, W1, b1, W2, b2):
    Bq, Sq, D = x.shape
    T = Bq * Sq
    E = NUM_EXPERTS
    H = HIDDEN_DIM
    xf = x.reshape(T, D)

    wcomb_t, aux = pl.pallas_call(
        _router_kernel,
        out_shape=(
            jax.ShapeDtypeStruct((E, T), jnp.float32),
            jax.ShapeDtypeStruct((1, 1), jnp.float32),
        ),
    )(xf, Wg)

    BT = 1024
    n_t = T // BT
    out = pl.pallas_call(
        _ffn_kernel,
        grid=(n_t, E),
        in_specs=[
            pl.BlockSpec((BT, D), lambda t, e: (t, 0)),
            pl.BlockSpec((1, H, D), lambda t, e: (e, 0, 0)),
            pl.BlockSpec((1, 1, H), lambda t, e: (e, 0, 0)),
            pl.BlockSpec((1, D, H), lambda t, e: (e, 0, 0)),
            pl.BlockSpec((1, 1, D), lambda t, e: (e, 0, 0)),
            pl.BlockSpec((NUM_EXPERTS, BT), lambda t, e: (0, t)),
        ],
        out_specs=pl.BlockSpec((BT, D), lambda t, e: (t, 0)),
        out_shape=jax.ShapeDtypeStruct((T, D), jnp.float32),
    )(xf.astype(jnp.bfloat16), W1.astype(jnp.bfloat16), b1.reshape(E, 1, H),
      W2.astype(jnp.bfloat16), b2.reshape(E, 1, D), wcomb_t)

    return out.reshape(Bq, Sq, D), aux.reshape(())


# speedup vs baseline: 1.3126x; 1.3126x over previous
import functools

import jax
import jax.numpy as jnp
from jax.experimental import pallas as pl

EMBED_DIM = 768
HIDDEN_DIM = 768
NUM_EXPERTS = 8
TOP_K = 2


def _router_kernel(x_ref, wg_ref, wcomb_ref, aux_ref):
    x = x_ref[...]
    wg = wg_ref[...]
    logits = jax.lax.dot_general(
        x, wg, (((1,), (1,)), ((), ())), preferred_element_type=jnp.float32
    )
    m = jnp.max(logits, axis=-1, keepdims=True)
    ex = jnp.exp(logits - m)
    probs = ex / jnp.sum(ex, axis=-1, keepdims=True)

    T, E = probs.shape
    idx = jax.lax.broadcasted_iota(jnp.int32, (T, E), 1)
    big = jnp.int32(E)
    m1 = jnp.max(probs, axis=-1, keepdims=True)
    i1 = jnp.min(jnp.where(probs == m1, idx, big), axis=-1, keepdims=True)
    masked = jnp.where(idx == i1, -jnp.inf, probs)
    m2 = jnp.max(masked, axis=-1, keepdims=True)
    i2 = jnp.min(jnp.where(masked == m2, idx, big), axis=-1, keepdims=True)

    denom = m1 + m2
    g1 = m1 / denom
    g2 = m2 / denom

    onehot1 = (idx == i1).astype(jnp.float32)
    onehot2 = (idx == i2).astype(jnp.float32)
    wcomb = g1 * onehot1 + g2 * onehot2
    wcomb_ref[...] = jnp.transpose(wcomb)

    f = jnp.sum(onehot1 + onehot2, axis=0) / jnp.float32(T)
    p = jnp.sum(probs, axis=0) / jnp.float32(T)
    aux_ref[...] = (jnp.float32(NUM_EXPERTS) * jnp.sum(f * p)).reshape(1, 1)


def _ffn_kernel(x_ref, w1_ref, b1_ref, w2_ref, b2_ref, wt_ref, out_ref):
    e = pl.program_id(1)

    @pl.when(e == 0)
    def _init():
        out_ref[...] = jnp.zeros_like(out_ref)

    xb = x_ref[...]
    w1 = w1_ref[0]
    h = jax.lax.dot_general(
        xb, w1, (((1,), (1,)), ((), ())), preferred_element_type=jnp.float32
    ) + b1_ref[0]
    h = h * jax.nn.sigmoid(h)
    w2 = w2_ref[0]
    eo = jax.lax.dot_general(
        h.astype(w2.dtype), w2, (((1,), (1,)), ((), ())),
        preferred_element_type=jnp.float32,
    ) + b2_ref[0]
    wcol = jnp.transpose(wt_ref[pl.ds(e, 1), :])
    out_ref[...] += wcol * eo


def kernel(x, Wg, W1, b1, W2, b2):
    Bq, Sq, D = x.shape
    T = Bq * Sq
    E = NUM_EXPERTS
    H = HIDDEN_DIM
    xf = x.reshape(T, D)

    wcomb_t, aux = pl.pallas_call(
        _router_kernel,
        out_shape=(
            jax.ShapeDtypeStruct((E, T), jnp.float32),
            jax.ShapeDtypeStruct((1, 1), jnp.float32),
        ),
    )(xf, Wg)

    BT = 1024
    n_t = T // BT
    out = pl.pallas_call(
        _ffn_kernel,
        grid=(n_t, E),
        in_specs=[
            pl.BlockSpec((BT, D), lambda t, e: (t, 0)),
            pl.BlockSpec((1, H, D), lambda t, e: (e, 0, 0)),
            pl.BlockSpec((1, 1, H), lambda t, e: (e, 0, 0)),
            pl.BlockSpec((1, D, H), lambda t, e: (e, 0, 0)),
            pl.BlockSpec((1, 1, D), lambda t, e: (e, 0, 0)),
            pl.BlockSpec((NUM_EXPERTS, BT), lambda t, e: (0, t)),
        ],
        out_specs=pl.BlockSpec((BT, D), lambda t, e: (t, 0)),
        out_shape=jax.ShapeDtypeStruct((T, D), jnp.float32),
    )(xf, W1, b1.reshape(E, 1, H), W2, b2.reshape(E, 1, D), wcomb_t)

    return out.reshape(Bq, Sq, D), aux.reshape(())


# speedup vs baseline: 1.3348x; 1.0169x over previous
import functools

import jax
import jax.numpy as jnp
from jax.experimental import pallas as pl

EMBED_DIM = 768
HIDDEN_DIM = 768
NUM_EXPERTS = 8
TOP_K = 2


def _router_kernel(x_ref, wg_ref, wcomb_ref, aux_ref):
    x = x_ref[...]
    wg = wg_ref[...]
    logits = jax.lax.dot_general(
        x, wg, (((1,), (1,)), ((), ())), preferred_element_type=jnp.float32
    )
    m = jnp.max(logits, axis=-1, keepdims=True)
    ex = jnp.exp(logits - m)
    probs = ex / jnp.sum(ex, axis=-1, keepdims=True)

    T, E = probs.shape
    idx = jax.lax.broadcasted_iota(jnp.int32, (T, E), 1)
    big = jnp.int32(E)
    m1 = jnp.max(probs, axis=-1, keepdims=True)
    i1 = jnp.min(jnp.where(probs == m1, idx, big), axis=-1, keepdims=True)
    masked = jnp.where(idx == i1, -jnp.inf, probs)
    m2 = jnp.max(masked, axis=-1, keepdims=True)
    i2 = jnp.min(jnp.where(masked == m2, idx, big), axis=-1, keepdims=True)

    denom = m1 + m2
    g1 = m1 / denom
    g2 = m2 / denom

    onehot1 = (idx == i1).astype(jnp.float32)
    onehot2 = (idx == i2).astype(jnp.float32)
    wcomb = g1 * onehot1 + g2 * onehot2
    wcomb_ref[...] = jnp.transpose(wcomb)

    f = jnp.sum(onehot1 + onehot2, axis=0) / jnp.float32(T)
    p = jnp.sum(probs, axis=0) / jnp.float32(T)
    aux_ref[...] = (jnp.float32(NUM_EXPERTS) * jnp.sum(f * p)).reshape(1, 1)


def _ffn_kernel(x_ref, w1_ref, b1_ref, w2_ref, b2_ref, wt_ref, out_ref):
    e = pl.program_id(1)

    @pl.when(e == 0)
    def _init():
        out_ref[...] = jnp.zeros_like(out_ref)

    xb = x_ref[...].astype(jnp.bfloat16)
    w1 = w1_ref[0].astype(jnp.bfloat16)
    h = jax.lax.dot_general(
        xb, w1, (((1,), (1,)), ((), ())), preferred_element_type=jnp.float32
    ) + b1_ref[0]
    h = h * jax.nn.sigmoid(h)
    w2 = w2_ref[0].astype(jnp.bfloat16)
    eo = jax.lax.dot_general(
        h.astype(jnp.bfloat16), w2, (((1,), (1,)), ((), ())),
        preferred_element_type=jnp.float32,
    ) + b2_ref[0]
    wcol = jnp.transpose(wt_ref[pl.ds(e, 1), :])
    out_ref[...] += wcol * eo


def kernel(x, Wg, W1, b1, W2, b2):
    Bq, Sq, D = x.shape
    T = Bq * Sq
    E = NUM_EXPERTS
    H = HIDDEN_DIM
    xf = x.reshape(T, D)

    wcomb_t, aux = pl.pallas_call(
        _router_kernel,
        out_shape=(
            jax.ShapeDtypeStruct((E, T), jnp.float32),
            jax.ShapeDtypeStruct((1, 1), jnp.float32),
        ),
    )(xf, Wg)

    BT = 1024
    n_t = T // BT
    out = pl.pallas_call(
        _ffn_kernel,
        grid=(n_t, E),
        in_specs=[
            pl.BlockSpec((BT, D), lambda t, e: (t, 0)),
            pl.BlockSpec((1, H, D), lambda t, e: (e, 0, 0)),
            pl.BlockSpec((1, 1, H), lambda t, e: (e, 0, 0)),
            pl.BlockSpec((1, D, H), lambda t, e: (e, 0, 0)),
            pl.BlockSpec((1, 1, D), lambda t, e: (e, 0, 0)),
            pl.BlockSpec((NUM_EXPERTS, BT), lambda t, e: (0, t)),
        ],
        out_specs=pl.BlockSpec((BT, D), lambda t, e: (t, 0)),
        out_shape=jax.ShapeDtypeStruct((T, D), jnp.float32),
    )(xf, W1, b1.reshape(E, 1, H), W2, b2.reshape(E, 1, D), wcomb_t)

    return out.reshape(Bq, Sq, D), aux.reshape(())
